# baseline (speedup 1.0000x reference)
import functools

import jax
import jax.numpy as jnp
from jax import lax
from jax.experimental import pallas as pl
from jax.experimental.pallas import tpu as pltpu
from jax.experimental.pallas import tpu_sc as plsc

B = 2
D = H = W = 128
V = D * H * W
LANES = 16
NC, NS = 2, 16
NW = NC * NS
DPW = D // (NW // B)

DBLK, HBLK = 4, 8
NVOX = DBLK * HBLK * W
STEPS = NVOX // LANES
HALO_Z = 6
HALO_Y = 5
HALO_CHK = 5.0
ZWIN = DBLK + 2 * HALO_Z + 2
YWIN = HBLK + 2 * HALO_Y + 2
SLAB = ZWIN * YWIN * W
CPD = H // HBLK
NCHUNK = (DPW // DBLK) * CPD
NSUB = 512
SSTEPS = NSUB // LANES

f32 = jnp.float32
i32 = jnp.int32


def _floor_frac(pos_i, f):
    pos = pos_i.astype(f32) + f
    t = pos.astype(i32)
    t = t - jnp.where(t.astype(f32) > pos, 1, 0)
    return t, pos - t.astype(f32)


def _tec_body(src_hbm, src3_hbm, fz_hbm, fy_hbm, fx_hbm, out_hbm,
              slab_a, slab_b, fz_a, fz_b, fy_a, fy_b, fx_a, fx_b,
              acc_a, acc_b, idx2, val2,
              sem_a, sem_b, sem_g, sem_oa, sem_ob):
    cid = lax.axis_index("c")
    sid = lax.axis_index("s")
    wid = sid * NC + cid
    batch = wid // (NW // B)
    bofs = batch * V
    bofs_d = batch * D
    d_base = (wid % (NW // B)) * DPW
    iota = lax.iota(i32, LANES)

    def tile_geom(c):
        d0 = d_base + (c // CPD) * DBLK
        h0 = (c % CPD) * HBLK
        z_base = jnp.clip(d0 - HALO_Z, 0, D - ZWIN)
        y_base = jnp.clip(h0 - HALO_Y, 0, H - YWIN)
        return d0, h0, z_base, y_base

    def issue(c, slab, fzv, fyv, fxv, sem):
        d0, h0, z_base, y_base = tile_geom(jnp.minimum(c, NCHUNK - 1))
        pltpu.async_copy(
            src3_hbm.at[pl.ds(bofs_d + z_base, ZWIN), pl.ds(y_base, YWIN), :],
            slab, sem)
        tsl = (pl.ds(bofs_d + d0, DBLK), pl.ds(h0, HBLK), slice(None))
        pltpu.async_copy(fz_hbm.at[tsl], fzv, sem)
        pltpu.async_copy(fy_hbm.at[tsl], fyv, sem)
        pltpu.async_copy(fx_hbm.at[tsl], fxv, sem)

    def drain_in(slab, fzv, fyv, fxv, sem):
        tsl = (pl.ds(bofs_d, DBLK), pl.ds(0, HBLK), slice(None))
        pltpu.make_async_copy(
            src3_hbm.at[pl.ds(bofs_d, ZWIN), pl.ds(0, YWIN), :],
            slab, sem).wait()
        pltpu.make_async_copy(fz_hbm.at[tsl], fzv, sem).wait()
        pltpu.make_async_copy(fy_hbm.at[tsl], fyv, sem).wait()
        pltpu.make_async_copy(fx_hbm.at[tsl], fxv, sem).wait()

    def drain_out(accv, sem):
        pltpu.make_async_copy(
            accv,
            out_hbm.at[(pl.ds(bofs_d, DBLK), pl.ds(0, HBLK), slice(None))],
            sem).wait()

    def axis_zy(pos, lo_clip, win):
        pos = jnp.clip(pos, -1.0, 128.0)
        t = pos.astype(i32)
        t = t - jnp.where(t.astype(f32) > pos, 1, 0)
        t = jnp.minimum(t, D - 1)
        fr = pos - t.astype(f32)
        w0 = jnp.where(t >= 0, 1.0 - fr, 0.0)
        w1 = jnp.where(t <= D - 2, fr, 0.0)
        u = t - lo_clip
        c0 = jnp.clip(u, 0, win - 1)
        c1 = jnp.clip(u + 1, 0, win - 1)
        return c0, c1, w0, w1

    def axis_x(pos):
        pos = jnp.clip(pos, -1.0, 128.0)
        t = pos.astype(i32)
        t = t - jnp.where(t.astype(f32) > pos, 1, 0)
        t = jnp.minimum(t, W - 1)
        fr = pos - t.astype(f32)
        w0 = jnp.where(t >= 0, 1.0 - fr, 0.0)
        w1 = jnp.where(t <= W - 2, fr, 0.0)
        c0 = jnp.maximum(t, 0)
        c1 = jnp.minimum(t + 1, W - 1)
        return c0, c1, w0, w1

    def compute(c, slab, fzv, fyv, fxv, accv):
        d0, h0, z_base, y_base = tile_geom(c)
        def fast(i, mx):
            rr = i // (W // LANES)
            col = (i % (W // LANES)) * LANES
            di = rr // HBLK
            hi = rr % HBLK
            sl = (di, hi, pl.ds(col, LANES))
            hq = h0 + hi
            dq = d0 + di
            wq = col + iota

            fzx = fzv[sl]
            fyx = fyv[sl]
            fxx = fxv[sl]
            mx = jnp.maximum(mx, jnp.maximum(jnp.abs(fzx), jnp.abs(fyx)))

            zc0, zc1, wz0, wz1 = axis_zy(dq.astype(f32) + fzx, z_base, ZWIN)
            yc0, yc1, wy0, wy1 = axis_zy(hq.astype(f32) + fyx, y_base, YWIN)
            xp0, xp1, wx0, wx1 = axis_x(wq.astype(f32) + fxx)

            acc = None
            for zc, wz in ((zc0, wz0), (zc1, wz1)):
                for yc, wy in ((yc0, wy0), (yc1, wy1)):
                    v0 = plsc.load_gather(slab, [zc, yc, xp0])
                    v1 = plsc.load_gather(slab, [zc, yc, xp1])
                    inner = wx0 * v0 + wx1 * v1
                    term = (wz * wy) * inner
                    acc = term if acc is None else acc + term
            accv[sl] = acc
            return mx

        mx = plsc.parallel_loop(
            0, STEPS, 1, unroll=4, carry=jnp.zeros((LANES,), f32))(fast)

        @pl.when(jnp.max(mx) > HALO_CHK)
        def _slow():
            def sub_body(sub, carry2):
                first = True
                for zb in (0, 1):
                    for yb in (0, 1):
                        for xb in (0, 1):
                            def mkidx(i, carry1, zb=zb, yb=yb, xb=xb):
                                sl = pl.ds(i * LANES, LANES)
                                j = sub * NSUB + i * LANES + iota
                                wq = j & (W - 1)
                                rr = j >> 7
                                hq = h0 + (rr & (HBLK - 1))
                                dq = d0 + (rr >> 3)
                                row = sub * (NSUB // W) + (i >> 3)
                                fsl = (row // HBLK, row % HBLK,
                                       pl.ds((i & 7) * LANES, LANES))
                                z0, frz = _floor_frac(dq, fzv[fsl])
                                y0, fry = _floor_frac(hq, fyv[fsl])
                                x0, frx = _floor_frac(wq, fxv[fsl])
                                zi, yi, xi = z0 + zb, y0 + yb, x0 + xb
                                inb = ((zi >= 0) & (zi <= D - 1) &
                                       (yi >= 0) & (yi <= H - 1) &
                                       (xi >= 0) & (xi <= W - 1))
                                gz = jnp.clip(zi, 0, D - 1)
                                gy = jnp.clip(yi, 0, H - 1)
                                gx = jnp.clip(xi, 0, W - 1)
                                idx2[sl] = bofs + (gz << 14) + (gy << 7) + gx
                                return carry1

                            lax.fori_loop(0, SSTEPS, mkidx, 0)
                            pltpu.async_copy(src_hbm.at[idx2], val2, sem_g)

                            def accum(i, carry1, zb=zb, yb=yb, xb=xb,
                                      first=first):
                                sl = pl.ds(i * LANES, LANES)
                                row = sub * (NSUB // W) + (i >> 3)
                                osl = (row // HBLK, row % HBLK,
                                       pl.ds((i & 7) * LANES, LANES))
                                j = sub * NSUB + i * LANES + iota
                                wq = j & (W - 1)
                                rr = j >> 7
                                hq = h0 + (rr & (HBLK - 1))
                                dq = d0 + (rr >> 3)
                                z0, frz = _floor_frac(dq, fzv[osl])
                                y0, fry = _floor_frac(hq, fyv[osl])
                                x0, frx = _floor_frac(wq, fxv[osl])
                                zi, yi, xi = z0 + zb, y0 + yb, x0 + xb
                                inb = ((zi >= 0) & (zi <= D - 1) &
                                       (yi >= 0) & (yi <= H - 1) &
                                       (xi >= 0) & (xi <= W - 1))
                                wz = frz if zb else 1.0 - frz
                                wy = fry if yb else 1.0 - fry
                                wx = frx if xb else 1.0 - frx
                                wgt = jnp.where(inb, wz * wy * wx, 0.0)
                                contrib = wgt * val2[sl]
                                if first:
                                    accv[osl] = contrib
                                else:
                                    accv[osl] = accv[osl] + contrib
                                return carry1

                            pltpu.make_async_copy(
                                src_hbm.at[idx2], val2, sem_g).wait()
                            lax.fori_loop(0, SSTEPS, accum, 0)
                            first = False
                return carry2

            lax.fori_loop(0, NVOX // NSUB, sub_body, 0)

        pltpu.async_copy(
            accv,
            out_hbm.at[(pl.ds(bofs_d + d0, DBLK), pl.ds(h0, HBLK),
                        slice(None))],
            sem_oa if accv is acc_a else sem_ob)

    issue(jnp.int32(0), slab_a, fz_a, fy_a, fx_a, sem_a)

    def pipe(k, carry):
        ca = 2 * k
        cb = 2 * k + 1
        issue(cb, slab_b, fz_b, fy_b, fx_b, sem_b)
        drain_in(slab_a, fz_a, fy_a, fx_a, sem_a)

        @pl.when(k > 0)
        def _():
            drain_out(acc_a, sem_oa)

        compute(ca, slab_a, fz_a, fy_a, fx_a, acc_a)

        @pl.when(k < NCHUNK // 2 - 1)
        def _():
            issue(ca + 2, slab_a, fz_a, fy_a, fx_a, sem_a)

        drain_in(slab_b, fz_b, fy_b, fx_b, sem_b)

        @pl.when(k > 0)
        def _():
            drain_out(acc_b, sem_ob)

        compute(cb, slab_b, fz_b, fy_b, fx_b, acc_b)
        return carry

    lax.fori_loop(0, NCHUNK // 2, pipe, 0)
    drain_out(acc_a, sem_oa)
    drain_out(acc_b, sem_ob)


@jax.jit
def kernel(src, flow):
    src_flat = jnp.concatenate(
        [src.reshape(B * V), jnp.zeros((8,), jnp.float32)])
    src3 = src.reshape(B * D, H, W)
    fz = flow[:, 0].reshape(B * D, H, W)
    fy = flow[:, 1].reshape(B * D, H, W)
    fx = flow[:, 2].reshape(B * D, H, W)

    mesh = plsc.VectorSubcoreMesh(core_axis_name="c", subcore_axis_name="s")
    call = functools.partial(
        pl.kernel,
        out_type=jax.ShapeDtypeStruct((B * D, H, W), f32),
        mesh=mesh,
        compiler_params=pltpu.CompilerParams(
            needs_layout_passes=False, use_tc_tiling_on_sc=False),
        scratch_types=[
            pltpu.VMEM((ZWIN, YWIN, W), f32),
            pltpu.VMEM((ZWIN, YWIN, W), f32),
            pltpu.VMEM((DBLK, HBLK, W), f32),
            pltpu.VMEM((DBLK, HBLK, W), f32),
            pltpu.VMEM((DBLK, HBLK, W), f32),
            pltpu.VMEM((DBLK, HBLK, W), f32),
            pltpu.VMEM((DBLK, HBLK, W), f32),
            pltpu.VMEM((DBLK, HBLK, W), f32),
            pltpu.VMEM((DBLK, HBLK, W), f32),
            pltpu.VMEM((DBLK, HBLK, W), f32),
            pltpu.VMEM((NSUB,), i32),
            pltpu.VMEM((NSUB,), f32),
            pltpu.SemaphoreType.DMA,
            pltpu.SemaphoreType.DMA,
            pltpu.SemaphoreType.DMA,
            pltpu.SemaphoreType.DMA,
            pltpu.SemaphoreType.DMA,
        ],
    )(_tec_body)
    out = call(src_flat, src3, fz, fy, fx)
    return out.reshape(B, 1, D, H, W)


# speedup vs baseline: 1.3127x; 1.3127x over previous
import functools

import jax
import jax.numpy as jnp
from jax import lax
from jax.experimental import pallas as pl
from jax.experimental.pallas import tpu as pltpu
from jax.experimental.pallas import tpu_sc as plsc

B = 2
D = H = W = 128
V = D * H * W
LANES = 16
NC, NS = 2, 16
NW = NC * NS
DPW = D // (NW // B)

DBLK, HBLK = 4, 16
NVOX = DBLK * HBLK * W
STEPS = NVOX // LANES
HALO = 6
ZWIN = DBLK + 2 * HALO + 2
YWIN = HBLK + 2 * HALO + 2
SLAB = ZWIN * YWIN * W
CPD = H // HBLK
NCHUNK = (DPW // DBLK) * CPD
NSUB = 512
SSTEPS = NSUB // LANES

f32 = jnp.float32
i32 = jnp.int32


def _floor_frac(pos_i, f):
    pos = pos_i.astype(f32) + f
    t = pos.astype(i32)
    t = t - jnp.where(t.astype(f32) > pos, 1, 0)
    return t, pos - t.astype(f32)


def _tec_body(src_hbm, fz_hbm, fy_hbm, fx_hbm, out_hbm,
              slab, fz_a, fz_b, fy_a, fy_b, fx_a, fx_b,
              accv, idx2, val2,
              sem_s, sem_fa, sem_fb, sem_g, sem_o):
    cid = lax.axis_index("c")
    sid = lax.axis_index("s")
    wid = sid * NC + cid
    batch = wid // (NW // B)
    bofs = batch * V
    d_base = (wid % (NW // B)) * DPW
    iota = lax.iota(i32, LANES)

    def tile_geom(c):
        d0 = d_base + (c // CPD) * DBLK
        h0 = (c % CPD) * HBLK
        return d0, h0

    def issue_flow(c, fzv, fyv, fxv, sem):
        d0, h0 = tile_geom(jnp.minimum(c, NCHUNK - 1))
        for dd in range(DBLK):
            off = bofs + (d0 + dd) * (H * W) + h0 * W
            t = pl.ds(dd * HBLK * W, HBLK * W)
            pltpu.async_copy(fz_hbm.at[pl.ds(off, HBLK * W)], fzv.at[t], sem)
            pltpu.async_copy(fy_hbm.at[pl.ds(off, HBLK * W)], fyv.at[t], sem)
            pltpu.async_copy(fx_hbm.at[pl.ds(off, HBLK * W)], fxv.at[t], sem)

    def drain_flow(fzv, fyv, fxv, sem):
        for dd in range(DBLK):
            t = pl.ds(dd * HBLK * W, HBLK * W)
            pltpu.make_async_copy(fz_hbm.at[pl.ds(bofs, HBLK * W)],
                                  fzv.at[t], sem).wait()
            pltpu.make_async_copy(fy_hbm.at[pl.ds(bofs, HBLK * W)],
                                  fyv.at[t], sem).wait()
            pltpu.make_async_copy(fx_hbm.at[pl.ds(bofs, HBLK * W)],
                                  fxv.at[t], sem).wait()

    def drain_out():
        for dd in range(DBLK):
            pltpu.make_async_copy(
                accv.at[pl.ds(dd * HBLK * W, HBLK * W)],
                out_hbm.at[pl.ds(bofs + dd * (H * W), HBLK * W)],
                sem_o).wait()

    def axis_zy(pos, lo_clip, win):
        pos = jnp.clip(pos, -1.0, 128.0)
        t = pos.astype(i32)
        t = t - jnp.where(t.astype(f32) > pos, 1, 0)
        t = jnp.minimum(t, D - 1)
        fr = pos - t.astype(f32)
        w0 = jnp.where(t >= 0, 1.0 - fr, 0.0)
        w1 = jnp.where(t <= D - 2, fr, 0.0)
        u = t - lo_clip
        c0 = jnp.clip(u, 0, win - 1)
        c1 = jnp.clip(u + 1, 0, win - 1)
        return c0, c1, w0, w1

    def axis_x(pos):
        pos = jnp.clip(pos, -1.0, 128.0)
        t = pos.astype(i32)
        t = t - jnp.where(t.astype(f32) > pos, 1, 0)
        t = jnp.minimum(t, W - 1)
        fr = pos - t.astype(f32)
        w0 = jnp.where(t >= 0, 1.0 - fr, 0.0)
        w1 = jnp.where(t <= W - 2, fr, 0.0)
        c0 = jnp.maximum(t, 0)
        c1 = jnp.minimum(t + 1, W - 1)
        return c0, c1, w0, w1

    def chunk(c, fzv, fyv, fxv, sem_f, nfzv, nfyv, nfxv, nsem_f, first_c):
        d0, h0 = tile_geom(c)
        z_base = jnp.clip(d0 - HALO, 0, D - ZWIN)
        y_base = jnp.clip(h0 - HALO, 0, H - YWIN)

        issue_flow(c + 1, nfzv, nfyv, nfxv, nsem_f)

        for zz in range(ZWIN):
            off = bofs + (z_base + zz) * (H * W) + y_base * W
            pltpu.async_copy(
                src_hbm.at[pl.ds(off, YWIN * W)],
                slab.at[pl.ds(zz * YWIN * W, YWIN * W)], sem_s)

        @pl.when(jnp.logical_not(first_c))
        def _():
            drain_out()

        for zz in range(ZWIN):
            pltpu.make_async_copy(
                src_hbm.at[pl.ds(bofs, YWIN * W)],
                slab.at[pl.ds(zz * YWIN * W, YWIN * W)], sem_s).wait()
        drain_flow(fzv, fyv, fxv, sem_f)

        def fast(i, mx):
            sl = pl.ds(i * LANES, LANES)
            rr = i // (W // LANES)
            hq = h0 + (rr % HBLK)
            dq = d0 + rr // HBLK
            wq = (i % (W // LANES)) * LANES + iota

            fzx = fzv[sl]
            fyx = fyv[sl]
            fxx = fxv[sl]
            mx = jnp.maximum(mx, jnp.maximum(jnp.abs(fzx), jnp.abs(fyx)))

            zc0, zc1, wz0, wz1 = axis_zy(dq.astype(f32) + fzx, z_base, ZWIN)
            yc0, yc1, wy0, wy1 = axis_zy(hq.astype(f32) + fyx, y_base, YWIN)
            xp0, xp1, wx0, wx1 = axis_x(wq.astype(f32) + fxx)

            zp0 = zc0 * (YWIN * W)
            zp1 = zc1 * (YWIN * W)
            yp0 = yc0 << 7
            yp1 = yc1 << 7

            acc = None
            for zp, wz in ((zp0, wz0), (zp1, wz1)):
                for yp, wy in ((yp0, wy0), (yp1, wy1)):
                    bzy = zp + yp
                    v0 = plsc.load_gather(slab, [bzy + xp0])
                    v1 = plsc.load_gather(slab, [bzy + xp1])
                    inner = wx0 * v0 + wx1 * v1
                    term = (wz * wy) * inner
                    acc = term if acc is None else acc + term
            accv[sl] = acc
            return mx

        mx = plsc.parallel_loop(
            0, STEPS, 1, unroll=4, carry=jnp.zeros((LANES,), f32))(fast)

        @pl.when(jnp.max(mx) > jnp.float32(HALO))
        def _slow():
            def sub_body(sub, carry2):
                first = True
                for zb in (0, 1):
                    for yb in (0, 1):
                        for xb in (0, 1):
                            def corner(i, zb=zb, yb=yb, xb=xb):
                                fsl = pl.ds(sub * NSUB + i * LANES, LANES)
                                j = sub * NSUB + i * LANES + iota
                                wq = j & (W - 1)
                                rr = j >> 7
                                hq = h0 + (rr & (HBLK - 1))
                                dq = d0 + (rr >> 4)
                                z0, frz = _floor_frac(dq, fzv[fsl])
                                y0, fry = _floor_frac(hq, fyv[fsl])
                                x0, frx = _floor_frac(wq, fxv[fsl])
                                zi, yi, xi = z0 + zb, y0 + yb, x0 + xb
                                inb = ((zi >= 0) & (zi <= D - 1) &
                                       (yi >= 0) & (yi <= H - 1) &
                                       (xi >= 0) & (xi <= W - 1))
                                wz = frz if zb else 1.0 - frz
                                wy = fry if yb else 1.0 - fry
                                wx = frx if xb else 1.0 - frx
                                wgt = jnp.where(inb, wz * wy * wx, 0.0)
                                gz = jnp.clip(zi, 0, D - 1)
                                gy = jnp.clip(yi, 0, H - 1)
                                gx = jnp.clip(xi, 0, W - 1)
                                gidx = bofs + (gz << 14) + (gy << 7) + gx
                                return gidx, wgt

                            def mkidx(i, carry1, corner=corner):
                                idx2[pl.ds(i * LANES, LANES)] = corner(i)[0]
                                return carry1

                            lax.fori_loop(0, SSTEPS, mkidx, 0)
                            pltpu.async_copy(src_hbm.at[idx2], val2, sem_g)

                            def accum(i, carry1, corner=corner, first=first):
                                sl = pl.ds(i * LANES, LANES)
                                osl = pl.ds(sub * NSUB + i * LANES, LANES)
                                contrib = corner(i)[1] * val2[sl]
                                if first:
                                    accv[osl] = contrib
                                else:
                                    accv[osl] = accv[osl] + contrib
                                return carry1

                            pltpu.make_async_copy(
                                src_hbm.at[idx2], val2, sem_g).wait()
                            lax.fori_loop(0, SSTEPS, accum, 0)
                            first = False
                return carry2

            lax.fori_loop(0, NVOX // NSUB, sub_body, 0)

        for dd in range(DBLK):
            off = bofs + (d0 + dd) * (H * W) + h0 * W
            pltpu.async_copy(
                accv.at[pl.ds(dd * HBLK * W, HBLK * W)],
                out_hbm.at[pl.ds(off, HBLK * W)], sem_o)

    issue_flow(jnp.int32(0), fz_a, fy_a, fx_a, sem_fa)

    def pipe(k, carry):
        ca = 2 * k
        cb = 2 * k + 1
        chunk(ca, fz_a, fy_a, fx_a, sem_fa,
              fz_b, fy_b, fx_b, sem_fb, k == 0)
        chunk(cb, fz_b, fy_b, fx_b, sem_fb,
              fz_a, fy_a, fx_a, sem_fa, False)
        return carry

    lax.fori_loop(0, NCHUNK // 2, pipe, 0)
    drain_out()
    drain_flow(fz_a, fy_a, fx_a, sem_fa)


@jax.jit
def kernel(src, flow):
    src_flat = src.reshape(B * V)
    fz = flow[:, 0].reshape(B * V)
    fy = flow[:, 1].reshape(B * V)
    fx = flow[:, 2].reshape(B * V)

    mesh = plsc.VectorSubcoreMesh(core_axis_name="c", subcore_axis_name="s")
    call = functools.partial(
        pl.kernel,
        out_type=jax.ShapeDtypeStruct((B * V,), f32),
        mesh=mesh,
        compiler_params=pltpu.CompilerParams(needs_layout_passes=False),
        scratch_types=[
            pltpu.VMEM((SLAB,), f32),
            pltpu.VMEM((NVOX,), f32),
            pltpu.VMEM((NVOX,), f32),
            pltpu.VMEM((NVOX,), f32),
            pltpu.VMEM((NVOX,), f32),
            pltpu.VMEM((NVOX,), f32),
            pltpu.VMEM((NVOX,), f32),
            pltpu.VMEM((NVOX,), f32),
            pltpu.VMEM((NSUB,), i32),
            pltpu.VMEM((NSUB,), f32),
            pltpu.SemaphoreType.DMA,
            pltpu.SemaphoreType.DMA,
            pltpu.SemaphoreType.DMA,
            pltpu.SemaphoreType.DMA,
            pltpu.SemaphoreType.DMA,
        ],
    )(_tec_body)
    out = call(src_flat, fz, fy, fx)
    return out.reshape(B, 1, D, H, W)
